# baseline (speedup 1.0000x reference)
import functools

import jax
import jax.numpy as jnp
from jax import lax
from jax.experimental import pallas as pl
from jax.experimental.pallas import tpu as pltpu
from jax.experimental.pallas import tpu_sc as plsc

N_ROWS = 16384
N_COLS = 126
COL0 = 100
N_OUT = 26

SC_ROWS = 4096
TC_ROWS = N_ROWS - SC_ROWS
TC_BLK = 2048

_INFO = plsc.get_sparse_core_info()
_NC = _INFO.num_cores
_NS = _INFO.num_subcores
_NW = _NC * _NS
_ROWS_PER = SC_ROWS // _NW
_NCHUNK = 2
_CH = _ROWS_PER // _NCHUNK


def _sc_body(in_hbm, out_hbm, ibuf, obuf, isems, osems):
    wid = lax.axis_index("s") * _NC + lax.axis_index("c")
    base = wid * _ROWS_PER

    in_copies = []
    for k in range(_NCHUNK):
        in_copies.append(
            pltpu.async_copy(
                in_hbm.at[pl.ds(base + k * _CH, _CH), :], ibuf.at[k], isems.at[k]
            )
        )

    out_copies = []
    for k in range(_NCHUNK):
        in_copies[k].wait()

        def realign(row, _, k=k):
            v0 = ibuf[k, row, pl.ds(COL0, 16)]
            v1 = ibuf[k, row, pl.ds(COL0 + N_OUT - 16, 16)]
            obuf[k, row, pl.ds(0, 16)] = v0
            obuf[k, row, pl.ds(N_OUT - 16, 16)] = v1
            return 0

        lax.fori_loop(0, _CH, realign, 0, unroll=8)
        out_copies.append(
            pltpu.async_copy(
                obuf.at[k], out_hbm.at[pl.ds(base + k * _CH, _CH), :], osems.at[k]
            )
        )
    for c in out_copies:
        c.wait()


def _sc_part(inputs):
    mesh = plsc.VectorSubcoreMesh(core_axis_name="c", subcore_axis_name="s")
    k = pl.kernel(
        _sc_body,
        mesh=mesh,
        out_type=jax.ShapeDtypeStruct((SC_ROWS, N_OUT), jnp.float32),
        scratch_types=[
            pltpu.VMEM((_NCHUNK, _CH, N_COLS), jnp.float32),
            pltpu.VMEM((_NCHUNK, _CH, N_OUT), jnp.float32),
            pltpu.SemaphoreType.DMA((_NCHUNK,)),
            pltpu.SemaphoreType.DMA((_NCHUNK,)),
        ],
    )
    return k(inputs)


def _tc_body(i_ref, o_ref):
    o_ref[...] = i_ref[:, COL0:COL0 + N_OUT]


def _tc_part(inputs):
    return pl.pallas_call(
        _tc_body,
        grid=(TC_ROWS // TC_BLK,),
        in_specs=[
            pl.BlockSpec((TC_BLK, N_COLS), lambda i: (i + SC_ROWS // TC_BLK, 0))
        ],
        out_specs=pl.BlockSpec((TC_BLK, N_OUT), lambda i: (i, 0)),
        out_shape=jax.ShapeDtypeStruct((TC_ROWS, N_OUT), jnp.float32),
    )(inputs)


@jax.jit
def kernel(inputs):
    sc_out = _sc_part(inputs)
    tc_out = _tc_part(inputs)
    return jnp.concatenate([sc_out, tc_out], axis=0)
